# baseline (speedup 1.0000x reference)
import functools

import numpy as np
import jax
import jax.numpy as jnp
from jax import lax
from jax.experimental import pallas as pl
from jax.experimental.pallas import tpu as pltpu
from jax.experimental.pallas import tpu_sc as plsc

HID = 128
NL = 3
N = 10000
NT = 10240
NSTRIPE = NT // 16
ZROWS = 16
CHUNK = 64
IBLK = 16
NBUF = 4
F32 = jnp.float32
I32 = jnp.int32
BN_SCALE = float(1.0 / np.sqrt(1.0 + 1e-5))


def _ceil_to(x, m):
    return ((x + m - 1) // m) * m


def _enc_body(x_ref, w_ref, b_ref, o_ref, g_ref):
    i = pl.program_id(0)
    h = jnp.dot(x_ref[...], w_ref[...], preferred_element_type=F32) + b_ref[...]
    h = jnp.maximum(h, 0.0)
    o_ref[...] = h
    cm = jnp.max(h, axis=0, keepdims=True)

    @pl.when(i == 0)
    def _():
        g_ref[...] = jnp.zeros((8, HID), F32)

    g_ref[...] = jnp.maximum(g_ref[...], jnp.broadcast_to(cm, (8, HID)))


def _encode(x, p):
    blk = 2048
    nin = x.shape[1]
    return pl.pallas_call(
        _enc_body,
        grid=(NT // blk,),
        in_specs=[
            pl.BlockSpec((blk, nin), lambda i: (i, 0)),
            pl.BlockSpec((nin, HID), lambda i: (0, 0)),
            pl.BlockSpec((HID,), lambda i: (0,)),
        ],
        out_specs=[
            pl.BlockSpec((blk, HID), lambda i: (i, 0)),
            pl.BlockSpec((8, HID), lambda i: (0, 0)),
        ],
        out_shape=[
            jax.ShapeDtypeStruct((NT, HID), F32),
            jax.ShapeDtypeStruct((8, HID), F32),
        ],
    )(x, p["W"], p["b"])


def _prep_body(x_ref, g_ref, t_ref):
    m = jnp.maximum(x_ref[...], 0.0) + 1e-7
    G = jnp.maximum(g_ref[0:1, :], 0.0) + 1e-7
    P = jnp.exp(m - G)
    Q = m * P
    t_ref[0] = jnp.concatenate([P[:, :64], Q[:, :64]], axis=1)
    t_ref[1] = jnp.concatenate([P[:, 64:], Q[:, 64:]], axis=1)


def _prep_tables(x, gmax):
    blk = 2048
    return pl.pallas_call(
        _prep_body,
        grid=(NT // blk,),
        in_specs=[
            pl.BlockSpec((blk, HID), lambda i: (i, 0)),
            pl.BlockSpec((8, HID), lambda i: (0, 0)),
        ],
        out_specs=pl.BlockSpec((2, blk, HID), lambda i: (0, i, 0)),
        out_shape=jax.ShapeDtypeStruct((2, NT, HID), F32),
    )(x, gmax)


def _combine_body(a1_ref, a2_ref, x_ref,
                  w11, b11, w12, b12, w21, b21, w22, b22,
                  h2_ref, xn_ref, g_ref, *, norm):
    x = x_ref[...]

    def conv(a_ref, w1, b1, w2, b2):
        c0 = a_ref[0]
        c1 = a_ref[1]
        den = jnp.concatenate([c0[:, :64], c1[:, :64]], axis=1)
        num = jnp.concatenate([c0[:, 64:], c1[:, 64:]], axis=1)
        aggr = jnp.where(den > 0, num / den, 0.0)
        o = aggr + x
        h = jnp.dot(o, w1[...], preferred_element_type=F32) + b1[...]
        if norm == "layer":
            mu = jnp.mean(h, axis=1, keepdims=True)
            var = jnp.mean((h - mu) ** 2, axis=1, keepdims=True)
            h = (h - mu) / jnp.sqrt(var + 1e-5)
        else:
            h = h * BN_SCALE
        h = jnp.maximum(h, 0.0)
        return jnp.dot(h, w2[...], preferred_element_type=F32) + b2[...]

    g1 = conv(a1_ref, w11, b11, w12, b12)
    g2 = conv(a2_ref, w21, b21, w22, b22)
    h2 = 0.5 * (g1 + g2)
    h2_ref[...] = h2
    xn = 0.5 * (jnp.maximum(h2, 0.0) + x)
    xn_ref[...] = xn
    i = pl.program_id(0)

    @pl.when(i == 0)
    def _():
        g_ref[...] = jnp.zeros((8, HID), F32)

    g_ref[...] = jnp.maximum(
        g_ref[...], jnp.broadcast_to(jnp.max(xn, axis=0, keepdims=True), (8, HID)))


def _combine(acc1, acc2, x, p1, p2, norm):
    blk = 2048
    body = functools.partial(_combine_body, norm=norm)
    wspec = lambda shp: pl.BlockSpec(shp, lambda i: tuple(0 for _ in shp))
    return pl.pallas_call(
        body,
        grid=(NT // blk,),
        in_specs=[
            pl.BlockSpec((2, blk, HID), lambda i: (0, i, 0)),
            pl.BlockSpec((2, blk, HID), lambda i: (0, i, 0)),
            pl.BlockSpec((blk, HID), lambda i: (i, 0)),
            wspec((HID, 2 * HID)), wspec((2 * HID,)),
            wspec((2 * HID, HID)), wspec((HID,)),
            wspec((HID, 2 * HID)), wspec((2 * HID,)),
            wspec((2 * HID, HID)), wspec((HID,)),
        ],
        out_specs=[
            pl.BlockSpec((blk, HID), lambda i: (i, 0)),
            pl.BlockSpec((blk, HID), lambda i: (i, 0)),
            pl.BlockSpec((8, HID), lambda i: (0, 0)),
        ],
        out_shape=[
            jax.ShapeDtypeStruct((NT, HID), F32),
            jax.ShapeDtypeStruct((NT, HID), F32),
            jax.ShapeDtypeStruct((8, HID), F32),
        ],
    )(acc1, acc2, x,
      p1["l1"]["W"], p1["l1"]["b"], p1["l2"]["W"], p1["l2"]["b"],
      p2["l1"]["W"], p2["l1"]["b"], p2["l2"]["W"], p2["l2"]["b"])


def _pred_body(x_ref, w1_ref, b1_ref, w2_ref, b2_ref, o_ref):
    h = jnp.dot(x_ref[0], w1_ref[...], preferred_element_type=F32)
    h = jnp.maximum(h + b1_ref[...], 0.0)
    o = jnp.dot(h, w2_ref[...], preferred_element_type=F32)
    o_ref[0] = o + b2_ref[...]


def _pred_head(x, p1, p2, dout):
    blk = 2048
    return pl.pallas_call(
        _pred_body,
        grid=(NL, NT // blk),
        in_specs=[
            pl.BlockSpec((1, blk, HID), lambda i, j: (i, j, 0)),
            pl.BlockSpec((HID, HID), lambda i, j: (0, 0)),
            pl.BlockSpec((HID,), lambda i, j: (0,)),
            pl.BlockSpec((HID, dout), lambda i, j: (0, 0)),
            pl.BlockSpec((dout,), lambda i, j: (0,)),
        ],
        out_specs=pl.BlockSpec((1, blk, dout), lambda i, j: (i, j, 0)),
        out_shape=jax.ShapeDtypeStruct((NL, NT, dout), F32),
    )(x, p1["W"], p1["b"], p2["W"], p2["b"])


def _make_agg(rows_list, table_ids):
    nrel = len(rows_list)

    def body(*refs):
        tables = refs[0:3]
        idx = refs[3:3 + 3 * nrel]
        outs = refs[3 + 3 * nrel:3 + 4 * nrel]
        scr = refs[-(2 + 2 + NBUF + 1 + 1 + 2 * NBUF + 2):]
        sblks = scr[0:2]
        dblks = scr[2:4]
        bufs = scr[4:4 + NBUF]
        zbuf = scr[4 + NBUF]
        acc = scr[5 + NBUF]
        gsems = scr[6 + NBUF:6 + 2 * NBUF]
        ssems = scr[6 + 2 * NBUF:6 + 3 * NBUF]
        isems = scr[6 + 3 * NBUF:8 + 3 * NBUF]
        c = lax.axis_index("c")
        s = lax.axis_index("s")

        def zrow(i, _):
            for k in range(8):
                zbuf[i, pl.ds(k * 16, 16)] = jnp.zeros((16,), F32)
            return 0

        lax.fori_loop(0, ZROWS, zrow, 0)

        for r, rows in enumerate(rows_list):
            tbl = tables[table_ids[r]]
            s0, s1, d = idx[3 * r:3 * r + 3]
            out = outs[r]

            for z in range(NSTRIPE // ZROWS):
                pltpu.sync_copy(
                    zbuf, acc.at[pl.ds(s * NSTRIPE + z * ZROWS, ZROWS)])

            def stage(b, slot, sync):
                base = s * rows + b * IBLK
                if sync:
                    @pl.when(c == 0)
                    def _():
                        pltpu.sync_copy(s0.at[pl.ds(base, IBLK)], sblks[slot])

                    @pl.when(c == 1)
                    def _():
                        pltpu.sync_copy(s1.at[pl.ds(base, IBLK)], sblks[slot])

                    pltpu.sync_copy(d.at[pl.ds(base, IBLK)], dblks[slot])
                    return ()
                hc = [pltpu.async_copy(d.at[pl.ds(base, IBLK)], dblks[slot],
                                       isems[slot])]

                @pl.when(c == 0)
                def _():
                    pltpu.async_copy(s0.at[pl.ds(base, IBLK)], sblks[slot],
                                     isems[slot])

                @pl.when(c == 1)
                def _():
                    pltpu.async_copy(s1.at[pl.ds(base, IBLK)], sblks[slot],
                                     isems[slot])

                hc.append(pltpu.make_async_copy(
                    s0.at[pl.ds(base, IBLK)], sblks[slot], isems[slot]))
                return tuple(hc)

            def do_block(slot):
                sb = sblks[slot]
                db = dblks[slot]
                hg = [pltpu.async_copy(tbl.at[sb.at[0]], bufs[0], gsems[0])]
                hs = []
                for j in range(IBLK):
                    hg[j].wait()
                    if j + 1 < IBLK:
                        k = (j + 1) % NBUF
                        if j + 1 >= NBUF:
                            hs[j + 1 - NBUF].wait()
                        hg.append(pltpu.async_copy(
                            tbl.at[sb.at[j + 1]], bufs[k], gsems[k]))
                    hs.append(pltpu.async_copy(
                        bufs[j % NBUF], acc.at[db.at[j]],
                        ssems[j % NBUF], add=True))
                for j in range(max(0, IBLK - NBUF), IBLK):
                    hs[j].wait()

            stage(0, 0, True)
            plsc.subcore_barrier()

            def pair(bp, _):
                h1 = stage(2 * bp + 1, 1, False)
                do_block(0)
                h0 = stage(2 * bp + 2, 0, False)
                for h in h1:
                    h.wait()
                do_block(1)
                for h in h0:
                    h.wait()
                return 0

            nb = rows // IBLK
            lax.fori_loop(0, nb // 2, pair, 0)
            if nb % 2:
                do_block(0)
            plsc.subcore_barrier()

            pltpu.sync_copy(acc.at[pl.ds(s * NSTRIPE, NSTRIPE)],
                            out.at[pl.ds(c * NT + s * NSTRIPE, NSTRIPE)])

    mesh = plsc.VectorSubcoreMesh(core_axis_name="c", subcore_axis_name="s")
    return pl.kernel(
        body,
        out_type=[jax.ShapeDtypeStruct((2 * NT, HID), F32)
                  for _ in rows_list],
        mesh=mesh,
        scratch_types=(
            [pltpu.VMEM((IBLK, CHUNK), I32)] * 2
            + [pltpu.VMEM((IBLK, CHUNK), I32)] * 2
            + [pltpu.VMEM((CHUNK, HID), F32)] * NBUF
            + [pltpu.VMEM((ZROWS, HID), F32)]
            + [pltpu.VMEM_SHARED((NT, HID), F32)]
            + [pltpu.SemaphoreType.DMA] * NBUF
            + [pltpu.SemaphoreType.DMA] * NBUF
            + [pltpu.SemaphoreType.DMA] * 2
        ),
    )


def _pad_rows(x, n):
    return jnp.concatenate(
        [x, jnp.zeros((n - x.shape[0], x.shape[1]), x.dtype)], axis=0)


def _prep_edges(ei):
    e = ei.shape[1]
    ep = _ceil_to(e, 16 * IBLK * CHUNK)
    npad = ep - e
    src = ei[0].astype(I32)
    dst = ei[1].astype(I32)
    xtr = IBLK * CHUNK
    srcp = jnp.concatenate([src, jnp.full((npad + xtr,), N, I32)])
    dstp = jnp.concatenate(
        [dst, N + (jnp.arange(npad + xtr, dtype=I32) % (NT - N))])
    r = ep // CHUNK
    return (srcp.reshape(r + IBLK, CHUNK), (srcp + NT).reshape(r + IBLK, CHUNK),
            dstp.reshape(r + IBLK, CHUNK), r // 16)


def kernel(params, x_cons, x_vals, x_obj, ei_cons_to_vals, ei_vals_to_cons,
           ei_vals_to_obj, ei_obj_to_vals, ei_cons_to_obj, ei_obj_to_cons):
    rels = [
        ("cv", 0, ei_cons_to_vals),
        ("ov", 2, ei_obj_to_vals),
        ("vc", 1, ei_vals_to_cons),
        ("oc", 2, ei_obj_to_cons),
        ("vo", 1, ei_vals_to_obj),
        ("co", 0, ei_cons_to_obj),
    ]
    idx_arrays = []
    rows_list = []
    for _, _, ei in rels:
        *arrs, rows = _prep_edges(ei)
        idx_arrays += arrs
        rows_list.append(rows)
    table_ids = [t for _, t, _ in rels]
    agg = _make_agg(tuple(rows_list), tuple(table_ids))

    x = {
        "cons": _encode(_pad_rows(x_cons, NT), params["enc_cons"]),
        "vals": _encode(_pad_rows(x_vals, NT), params["enc_vals"]),
        "obj": _encode(_pad_rows(x_obj, NT), params["enc_obj"]),
    }

    def layer(carry, _):
        xs = carry
        tabs = [_prep_tables(*xs[t]).reshape(2 * NT, HID)
                for t in ("cons", "vals", "obj")]
        accs = agg(*tabs, *idx_arrays)
        accs = [a.reshape(2, NT, HID) for a in accs]
        h2v, xv, gv = _combine(accs[0], accs[1], xs["vals"][0],
                               params["conv_cv"], params["conv_ov"], "batch")
        h2c, xc, gc = _combine(accs[2], accs[3], xs["cons"][0],
                               params["conv_vc"], params["conv_oc"], "batch")
        h2o, xo, go = _combine(accs[4], accs[5], xs["obj"][0],
                               params["conv_vo"], params["conv_co"], "layer")
        new = {"cons": (xc, gc), "vals": (xv, gv), "obj": (xo, go)}
        return new, (h2c, h2v)

    _, (cons, vals) = lax.scan(layer, x, None, length=NL)
    vals = _pred_head(vals, params["pred_vals_1"], params["pred_vals_2"], 2)
    cons = _pred_head(cons, params["pred_cons_1"], params["pred_cons_2"], 1)
    vals_out = jnp.transpose(vals[:, :N, :], (1, 0, 2))
    cons_out = jnp.squeeze(cons[:, :N, :], axis=-1).T
    return vals_out, cons_out


# speedup vs baseline: 1.1235x; 1.1235x over previous
import functools

import numpy as np
import jax
import jax.numpy as jnp
from jax import lax
from jax.experimental import pallas as pl
from jax.experimental.pallas import tpu as pltpu
from jax.experimental.pallas import tpu_sc as plsc

HID = 128
NL = 3
N = 10000
NT = 10240
NSTRIPE = NT // 16
ZROWS = 64
CHUNK = 128
IBLK = 16
F32 = jnp.float32
I32 = jnp.int32
BN_SCALE = float(1.0 / np.sqrt(1.0 + 1e-5))


def _ceil_to(x, m):
    return ((x + m - 1) // m) * m


def _enc_body(x_ref, w_ref, b_ref, o_ref, g_ref):
    i = pl.program_id(0)
    h = jnp.dot(x_ref[...], w_ref[...], preferred_element_type=F32) + b_ref[...]
    h = jnp.maximum(h, 0.0)
    o_ref[...] = h
    cm = jnp.max(h, axis=0, keepdims=True)

    @pl.when(i == 0)
    def _():
        g_ref[...] = jnp.zeros((8, HID), F32)

    g_ref[...] = jnp.maximum(g_ref[...], jnp.broadcast_to(cm, (8, HID)))


def _encode(x, p):
    blk = 2048
    nin = x.shape[1]
    return pl.pallas_call(
        _enc_body,
        grid=(NT // blk,),
        in_specs=[
            pl.BlockSpec((blk, nin), lambda i: (i, 0)),
            pl.BlockSpec((nin, HID), lambda i: (0, 0)),
            pl.BlockSpec((HID,), lambda i: (0,)),
        ],
        out_specs=[
            pl.BlockSpec((blk, HID), lambda i: (i, 0)),
            pl.BlockSpec((8, HID), lambda i: (0, 0)),
        ],
        out_shape=[
            jax.ShapeDtypeStruct((NT, HID), F32),
            jax.ShapeDtypeStruct((8, HID), F32),
        ],
    )(x, p["W"], p["b"])


def _prep_body(x_ref, g_ref, t_ref):
    m = jnp.maximum(x_ref[...], 0.0) + 1e-7
    G = jnp.maximum(g_ref[0:1, :], 0.0) + 1e-7
    P = jnp.exp(m - G)
    Q = m * P
    t_ref[0] = jnp.concatenate([P[:, :64], Q[:, :64]], axis=1)
    t_ref[1] = jnp.concatenate([P[:, 64:], Q[:, 64:]], axis=1)


def _prep_tables(x, gmax):
    blk = 2048
    return pl.pallas_call(
        _prep_body,
        grid=(NT // blk,),
        in_specs=[
            pl.BlockSpec((blk, HID), lambda i: (i, 0)),
            pl.BlockSpec((8, HID), lambda i: (0, 0)),
        ],
        out_specs=pl.BlockSpec((2, blk, HID), lambda i: (0, i, 0)),
        out_shape=jax.ShapeDtypeStruct((2, NT, HID), F32),
    )(x, gmax)


def _combine_body(a1_ref, a2_ref, x_ref,
                  w11, b11, w12, b12, w21, b21, w22, b22,
                  h2_ref, xn_ref, g_ref, *, norm):
    x = x_ref[...]

    def conv(a_ref, w1, b1, w2, b2):
        c0 = a_ref[0]
        c1 = a_ref[1]
        den = jnp.concatenate([c0[:, :64], c1[:, :64]], axis=1)
        num = jnp.concatenate([c0[:, 64:], c1[:, 64:]], axis=1)
        aggr = jnp.where(den > 0, num / den, 0.0)
        o = aggr + x
        h = jnp.dot(o, w1[...], preferred_element_type=F32) + b1[...]
        if norm == "layer":
            mu = jnp.mean(h, axis=1, keepdims=True)
            var = jnp.mean((h - mu) ** 2, axis=1, keepdims=True)
            h = (h - mu) / jnp.sqrt(var + 1e-5)
        else:
            h = h * BN_SCALE
        h = jnp.maximum(h, 0.0)
        return jnp.dot(h, w2[...], preferred_element_type=F32) + b2[...]

    g1 = conv(a1_ref, w11, b11, w12, b12)
    g2 = conv(a2_ref, w21, b21, w22, b22)
    h2 = 0.5 * (g1 + g2)
    h2_ref[...] = h2
    xn = 0.5 * (jnp.maximum(h2, 0.0) + x)
    xn_ref[...] = xn
    i = pl.program_id(0)

    @pl.when(i == 0)
    def _():
        g_ref[...] = jnp.zeros((8, HID), F32)

    g_ref[...] = jnp.maximum(
        g_ref[...], jnp.broadcast_to(jnp.max(xn, axis=0, keepdims=True), (8, HID)))


def _combine(acc1, acc2, x, p1, p2, norm):
    blk = 2048
    body = functools.partial(_combine_body, norm=norm)
    wspec = lambda shp: pl.BlockSpec(shp, lambda i: tuple(0 for _ in shp))
    return pl.pallas_call(
        body,
        grid=(NT // blk,),
        in_specs=[
            pl.BlockSpec((2, blk, HID), lambda i: (0, i, 0)),
            pl.BlockSpec((2, blk, HID), lambda i: (0, i, 0)),
            pl.BlockSpec((blk, HID), lambda i: (i, 0)),
            wspec((HID, 2 * HID)), wspec((2 * HID,)),
            wspec((2 * HID, HID)), wspec((HID,)),
            wspec((HID, 2 * HID)), wspec((2 * HID,)),
            wspec((2 * HID, HID)), wspec((HID,)),
        ],
        out_specs=[
            pl.BlockSpec((blk, HID), lambda i: (i, 0)),
            pl.BlockSpec((blk, HID), lambda i: (i, 0)),
            pl.BlockSpec((8, HID), lambda i: (0, 0)),
        ],
        out_shape=[
            jax.ShapeDtypeStruct((NT, HID), F32),
            jax.ShapeDtypeStruct((NT, HID), F32),
            jax.ShapeDtypeStruct((8, HID), F32),
        ],
    )(acc1, acc2, x,
      p1["l1"]["W"], p1["l1"]["b"], p1["l2"]["W"], p1["l2"]["b"],
      p2["l1"]["W"], p2["l1"]["b"], p2["l2"]["W"], p2["l2"]["b"])


def _pred_body(x_ref, w1_ref, b1_ref, w2_ref, b2_ref, o_ref):
    h = jnp.dot(x_ref[0], w1_ref[...], preferred_element_type=F32)
    h = jnp.maximum(h + b1_ref[...], 0.0)
    o = jnp.dot(h, w2_ref[...], preferred_element_type=F32)
    o_ref[0] = o + b2_ref[...]


def _pred_head(x, p1, p2, dout):
    blk = 2048
    return pl.pallas_call(
        _pred_body,
        grid=(NL, NT // blk),
        in_specs=[
            pl.BlockSpec((1, blk, HID), lambda i, j: (i, j, 0)),
            pl.BlockSpec((HID, HID), lambda i, j: (0, 0)),
            pl.BlockSpec((HID,), lambda i, j: (0,)),
            pl.BlockSpec((HID, dout), lambda i, j: (0, 0)),
            pl.BlockSpec((dout,), lambda i, j: (0,)),
        ],
        out_specs=pl.BlockSpec((1, blk, dout), lambda i, j: (i, j, 0)),
        out_shape=jax.ShapeDtypeStruct((NL, NT, dout), F32),
    )(x, p1["W"], p1["b"], p2["W"], p2["b"])


def _make_agg(rows_list, table_ids, ntab):
    nrel = len(rows_list)

    def body(*refs):
        tables = refs[0:ntab]
        idx = refs[ntab:ntab + 3 * nrel]
        outs = refs[ntab + 3 * nrel:ntab + 4 * nrel]
        sblk, dblk, buf0, buf1, zbuf, acc, sem0, sem1 = refs[-8:]
        bufs = (buf0, buf1)
        sems = (sem0, sem1)
        c = lax.axis_index("c")
        s = lax.axis_index("s")

        def zrow(i, _):
            for k in range(8):
                zbuf[i, pl.ds(k * 16, 16)] = jnp.zeros((16,), F32)
            return 0

        lax.fori_loop(0, ZROWS, zrow, 0)

        for r, rows in enumerate(rows_list):
            tbl = tables[table_ids[r]]
            s0, s1, d = idx[3 * r:3 * r + 3]
            out = outs[r]

            for z in range(NSTRIPE // ZROWS):
                pltpu.sync_copy(
                    zbuf, acc.at[pl.ds(s * NSTRIPE + z * ZROWS, ZROWS)])
            plsc.subcore_barrier()

            def block(b, _):
                base = s * rows + b * IBLK

                @pl.when(c == 0)
                def _():
                    pltpu.sync_copy(s0.at[pl.ds(base, IBLK)], sblk)

                @pl.when(c == 1)
                def _():
                    pltpu.sync_copy(s1.at[pl.ds(base, IBLK)], sblk)

                pltpu.sync_copy(d.at[pl.ds(base, IBLK)], dblk)
                hs = [pltpu.async_copy(tbl.at[sblk.at[0]], bufs[0], sems[0])]
                for j in range(IBLK):
                    hs[j].wait()
                    if j + 1 < IBLK:
                        hs.append(pltpu.async_copy(
                            tbl.at[sblk.at[j + 1]],
                            bufs[(j + 1) % 2], sems[(j + 1) % 2]))
                    pltpu.sync_copy(bufs[j % 2], acc.at[dblk.at[j]], add=True)
                return 0

            lax.fori_loop(0, rows // IBLK, block, 0)
            plsc.subcore_barrier()

            pltpu.sync_copy(acc.at[pl.ds(s * NSTRIPE, NSTRIPE)],
                            out.at[pl.ds(c * NT + s * NSTRIPE, NSTRIPE)])

    mesh = plsc.VectorSubcoreMesh(core_axis_name="c", subcore_axis_name="s")
    return pl.kernel(
        body,
        out_type=[jax.ShapeDtypeStruct((2 * NT, HID), F32)
                  for _ in rows_list],
        mesh=mesh,
        scratch_types=[
            pltpu.VMEM((IBLK, CHUNK), I32),
            pltpu.VMEM((IBLK, CHUNK), I32),
            pltpu.VMEM((CHUNK, HID), F32),
            pltpu.VMEM((CHUNK, HID), F32),
            pltpu.VMEM((ZROWS, HID), F32),
            pltpu.VMEM_SHARED((NT, HID), F32),
            pltpu.SemaphoreType.DMA,
            pltpu.SemaphoreType.DMA,
        ],
    )


def _pad_rows(x, n):
    return jnp.concatenate(
        [x, jnp.zeros((n - x.shape[0], x.shape[1]), x.dtype)], axis=0)


def _prep_edges(ei):
    e = ei.shape[1]
    ep = _ceil_to(e, 16 * IBLK * CHUNK)
    npad = ep - e
    src = ei[0].astype(I32)
    dst = ei[1].astype(I32)
    srcp = jnp.concatenate([src, jnp.full((npad,), N, I32)])
    dstp = jnp.concatenate(
        [dst, N + (jnp.arange(npad, dtype=I32) % (NT - N))])
    r = ep // CHUNK
    return (srcp.reshape(r, CHUNK), (srcp + NT).reshape(r, CHUNK),
            dstp.reshape(r, CHUNK), r // 16)


def kernel(params, x_cons, x_vals, x_obj, ei_cons_to_vals, ei_vals_to_cons,
           ei_vals_to_obj, ei_obj_to_vals, ei_cons_to_obj, ei_obj_to_cons):
    groups = {
        "vals": [("cons", ei_cons_to_vals), ("obj", ei_obj_to_vals)],
        "cons": [("vals", ei_vals_to_cons), ("obj", ei_obj_to_cons)],
        "obj": [("vals", ei_vals_to_obj), ("cons", ei_cons_to_obj)],
    }
    aggs = {}
    gidx = {}
    gsrc = {}
    for t, grp in groups.items():
        idx_arrays = []
        rows_list = []
        for _, ei in grp:
            *arrs, rows = _prep_edges(ei)
            idx_arrays += arrs
            rows_list.append(rows)
        aggs[t] = _make_agg(tuple(rows_list), (0, 1), 2)
        gidx[t] = idx_arrays
        gsrc[t] = [srct for srct, _ in grp]

    x = {
        "cons": _encode(_pad_rows(x_cons, NT), params["enc_cons"]),
        "vals": _encode(_pad_rows(x_vals, NT), params["enc_vals"]),
        "obj": _encode(_pad_rows(x_obj, NT), params["enc_obj"]),
    }

    def layer(carry, _):
        xs = carry
        tabs = {t: _prep_tables(*xs[t]).reshape(2 * NT, HID)
                for t in ("cons", "vals", "obj")}

        def agg_for(t):
            a = aggs[t](*[tabs[s] for s in gsrc[t]], *gidx[t])
            return [x.reshape(2, NT, HID) for x in a]

        accV = agg_for("vals")
        accC = agg_for("cons")
        accO = agg_for("obj")
        h2v, xv, gv = _combine(accV[0], accV[1], xs["vals"][0],
                               params["conv_cv"], params["conv_ov"], "batch")
        h2c, xc, gc = _combine(accC[0], accC[1], xs["cons"][0],
                               params["conv_vc"], params["conv_oc"], "batch")
        h2o, xo, go = _combine(accO[0], accO[1], xs["obj"][0],
                               params["conv_vo"], params["conv_co"], "layer")
        new = {"cons": (xc, gc), "vals": (xv, gv), "obj": (xo, go)}
        return new, (h2c, h2v)

    _, (cons, vals) = lax.scan(layer, x, None, length=NL)
    vals = _pred_head(vals, params["pred_vals_1"], params["pred_vals_2"], 2)
    cons = _pred_head(cons, params["pred_cons_1"], params["pred_cons_2"], 1)
    vals_out = jnp.transpose(vals[:, :N, :], (1, 0, 2))
    cons_out = jnp.squeeze(cons[:, :N, :], axis=-1).T
    return vals_out, cons_out


# speedup vs baseline: 1.1421x; 1.0166x over previous
import functools

import numpy as np
import jax
import jax.numpy as jnp
from jax import lax
from jax.experimental import pallas as pl
from jax.experimental.pallas import tpu as pltpu
from jax.experimental.pallas import tpu_sc as plsc

HID = 128
NL = 3
N = 10000
NT = 10240
NSTRIPE = NT // 16
ZROWS = 64
CHUNK = 128
IBLK = 16
F32 = jnp.float32
I32 = jnp.int32
BN_SCALE = float(1.0 / np.sqrt(1.0 + 1e-5))


def _ceil_to(x, m):
    return ((x + m - 1) // m) * m


def _enc_body(x_ref, w_ref, b_ref, o_ref, g_ref):
    i = pl.program_id(0)
    h = jnp.dot(x_ref[...], w_ref[...], preferred_element_type=F32) + b_ref[...]
    h = jnp.maximum(h, 0.0)
    o_ref[...] = h
    cm = jnp.max(h, axis=0, keepdims=True)

    @pl.when(i == 0)
    def _():
        g_ref[...] = jnp.zeros((8, HID), F32)

    g_ref[...] = jnp.maximum(g_ref[...], jnp.broadcast_to(cm, (8, HID)))


def _encode(x, p):
    blk = 2048
    nin = x.shape[1]
    return pl.pallas_call(
        _enc_body,
        grid=(NT // blk,),
        in_specs=[
            pl.BlockSpec((blk, nin), lambda i: (i, 0)),
            pl.BlockSpec((nin, HID), lambda i: (0, 0)),
            pl.BlockSpec((HID,), lambda i: (0,)),
        ],
        out_specs=[
            pl.BlockSpec((blk, HID), lambda i: (i, 0)),
            pl.BlockSpec((8, HID), lambda i: (0, 0)),
        ],
        out_shape=[
            jax.ShapeDtypeStruct((NT, HID), F32),
            jax.ShapeDtypeStruct((8, HID), F32),
        ],
    )(x, p["W"], p["b"])


def _prep_body(x_ref, g_ref, t_ref):
    m = jnp.maximum(x_ref[...], 0.0) + 1e-7
    G = jnp.maximum(g_ref[0:1, :], 0.0) + 1e-7
    P = jnp.exp(m - G)
    Q = m * P
    t_ref[0] = jnp.concatenate([P[:, :64], Q[:, :64]], axis=1)
    t_ref[1] = jnp.concatenate([P[:, 64:], Q[:, 64:]], axis=1)


def _prep_tables(x, gmax):
    blk = 2048
    return pl.pallas_call(
        _prep_body,
        grid=(NT // blk,),
        in_specs=[
            pl.BlockSpec((blk, HID), lambda i: (i, 0)),
            pl.BlockSpec((8, HID), lambda i: (0, 0)),
        ],
        out_specs=pl.BlockSpec((2, blk, HID), lambda i: (0, i, 0)),
        out_shape=jax.ShapeDtypeStruct((2, NT, HID), F32),
    )(x, gmax)


def _combine_body(a1_ref, a2_ref, x_ref,
                  w11, b11, w12, b12, w21, b21, w22, b22,
                  h2_ref, xn_ref, g_ref, *, norm):
    x = x_ref[...]

    def conv(a_ref, w1, b1, w2, b2):
        c0 = a_ref[0]
        c1 = a_ref[1]
        den = jnp.concatenate([c0[:, :64], c1[:, :64]], axis=1)
        num = jnp.concatenate([c0[:, 64:], c1[:, 64:]], axis=1)
        aggr = jnp.where(den > 0, num / den, 0.0)
        o = aggr + x
        h = jnp.dot(o, w1[...], preferred_element_type=F32) + b1[...]
        if norm == "layer":
            mu = jnp.mean(h, axis=1, keepdims=True)
            var = jnp.mean((h - mu) ** 2, axis=1, keepdims=True)
            h = (h - mu) / jnp.sqrt(var + 1e-5)
        else:
            h = h * BN_SCALE
        h = jnp.maximum(h, 0.0)
        return jnp.dot(h, w2[...], preferred_element_type=F32) + b2[...]

    g1 = conv(a1_ref, w11, b11, w12, b12)
    g2 = conv(a2_ref, w21, b21, w22, b22)
    h2 = 0.5 * (g1 + g2)
    h2_ref[...] = h2
    xn = 0.5 * (jnp.maximum(h2, 0.0) + x)
    xn_ref[...] = xn
    i = pl.program_id(0)

    @pl.when(i == 0)
    def _():
        g_ref[...] = jnp.zeros((8, HID), F32)

    g_ref[...] = jnp.maximum(
        g_ref[...], jnp.broadcast_to(jnp.max(xn, axis=0, keepdims=True), (8, HID)))


def _combine(acc1, acc2, x, p1, p2, norm):
    blk = 2048
    body = functools.partial(_combine_body, norm=norm)
    wspec = lambda shp: pl.BlockSpec(shp, lambda i: tuple(0 for _ in shp))
    return pl.pallas_call(
        body,
        grid=(NT // blk,),
        in_specs=[
            pl.BlockSpec((2, blk, HID), lambda i: (0, i, 0)),
            pl.BlockSpec((2, blk, HID), lambda i: (0, i, 0)),
            pl.BlockSpec((blk, HID), lambda i: (i, 0)),
            wspec((HID, 2 * HID)), wspec((2 * HID,)),
            wspec((2 * HID, HID)), wspec((HID,)),
            wspec((HID, 2 * HID)), wspec((2 * HID,)),
            wspec((2 * HID, HID)), wspec((HID,)),
        ],
        out_specs=[
            pl.BlockSpec((blk, HID), lambda i: (i, 0)),
            pl.BlockSpec((blk, HID), lambda i: (i, 0)),
            pl.BlockSpec((8, HID), lambda i: (0, 0)),
        ],
        out_shape=[
            jax.ShapeDtypeStruct((NT, HID), F32),
            jax.ShapeDtypeStruct((NT, HID), F32),
            jax.ShapeDtypeStruct((8, HID), F32),
        ],
    )(acc1, acc2, x,
      p1["l1"]["W"], p1["l1"]["b"], p1["l2"]["W"], p1["l2"]["b"],
      p2["l1"]["W"], p2["l1"]["b"], p2["l2"]["W"], p2["l2"]["b"])


def _pred_body(x_ref, w1_ref, b1_ref, w2_ref, b2_ref, o_ref):
    h = jnp.dot(x_ref[0], w1_ref[...], preferred_element_type=F32)
    h = jnp.maximum(h + b1_ref[...], 0.0)
    o = jnp.dot(h, w2_ref[...], preferred_element_type=F32)
    o_ref[0] = o + b2_ref[...]


def _pred_head(x, p1, p2, dout):
    blk = 2048
    return pl.pallas_call(
        _pred_body,
        grid=(NL, NT // blk),
        in_specs=[
            pl.BlockSpec((1, blk, HID), lambda i, j: (i, j, 0)),
            pl.BlockSpec((HID, HID), lambda i, j: (0, 0)),
            pl.BlockSpec((HID,), lambda i, j: (0,)),
            pl.BlockSpec((HID, dout), lambda i, j: (0, 0)),
            pl.BlockSpec((dout,), lambda i, j: (0,)),
        ],
        out_specs=pl.BlockSpec((1, blk, dout), lambda i, j: (i, j, 0)),
        out_shape=jax.ShapeDtypeStruct((NL, NT, dout), F32),
    )(x, p1["W"], p1["b"], p2["W"], p2["b"])


def _make_agg(rows_list, table_ids):
    nrel = len(rows_list)

    def body(*refs):
        tables = refs[0:3]
        idx = refs[3:3 + 3 * nrel]
        outs = refs[3 + 3 * nrel:3 + 4 * nrel]
        sblk, dblk, buf0, buf1, zbuf, acc, sem0, sem1 = refs[-8:]
        bufs = (buf0, buf1)
        sems = (sem0, sem1)
        c = lax.axis_index("c")
        s = lax.axis_index("s")

        def zrow(i, _):
            for k in range(8):
                zbuf[i, pl.ds(k * 16, 16)] = jnp.zeros((16,), F32)
            return 0

        lax.fori_loop(0, ZROWS, zrow, 0)

        for r, rows in enumerate(rows_list):
            tbl = tables[table_ids[r]]
            s0, s1, d = idx[3 * r:3 * r + 3]
            out = outs[r]

            for z in range(NSTRIPE // ZROWS):
                pltpu.sync_copy(
                    zbuf, acc.at[pl.ds(s * NSTRIPE + z * ZROWS, ZROWS)])
            plsc.subcore_barrier()

            def block(b, _):
                base = s * rows + b * IBLK

                @pl.when(c == 0)
                def _():
                    pltpu.sync_copy(s0.at[pl.ds(base, IBLK)], sblk)

                @pl.when(c == 1)
                def _():
                    pltpu.sync_copy(s1.at[pl.ds(base, IBLK)], sblk)

                pltpu.sync_copy(d.at[pl.ds(base, IBLK)], dblk)
                hs = [pltpu.async_copy(tbl.at[sblk.at[0]], bufs[0], sems[0])]
                for j in range(IBLK):
                    hs[j].wait()
                    if j + 1 < IBLK:
                        hs.append(pltpu.async_copy(
                            tbl.at[sblk.at[j + 1]],
                            bufs[(j + 1) % 2], sems[(j + 1) % 2]))
                    pltpu.sync_copy(bufs[j % 2], acc.at[dblk.at[j]], add=True)
                return 0

            lax.fori_loop(0, rows // IBLK, block, 0)
            plsc.subcore_barrier()

            pltpu.sync_copy(acc.at[pl.ds(s * NSTRIPE, NSTRIPE)],
                            out.at[pl.ds(c * NT + s * NSTRIPE, NSTRIPE)])

    mesh = plsc.VectorSubcoreMesh(core_axis_name="c", subcore_axis_name="s")
    return pl.kernel(
        body,
        out_type=[jax.ShapeDtypeStruct((2 * NT, HID), F32)
                  for _ in rows_list],
        mesh=mesh,
        scratch_types=[
            pltpu.VMEM((IBLK, CHUNK), I32),
            pltpu.VMEM((IBLK, CHUNK), I32),
            pltpu.VMEM((CHUNK, HID), F32),
            pltpu.VMEM((CHUNK, HID), F32),
            pltpu.VMEM((ZROWS, HID), F32),
            pltpu.VMEM_SHARED((NT, HID), F32),
            pltpu.SemaphoreType.DMA,
            pltpu.SemaphoreType.DMA,
        ],
    )


def _pad_rows(x, n):
    return jnp.concatenate(
        [x, jnp.zeros((n - x.shape[0], x.shape[1]), x.dtype)], axis=0)


def _prep_edges(ei):
    e = ei.shape[1]
    ep = _ceil_to(e, 16 * IBLK * CHUNK)
    npad = ep - e
    src = ei[0].astype(I32)
    dst = ei[1].astype(I32)
    srcp = jnp.concatenate([src, jnp.full((npad,), N, I32)])
    dstp = jnp.concatenate(
        [dst, N + (jnp.arange(npad, dtype=I32) % (NT - N))])
    r = ep // CHUNK
    return (srcp.reshape(r, CHUNK), (srcp + NT).reshape(r, CHUNK),
            dstp.reshape(r, CHUNK), r // 16)


def kernel(params, x_cons, x_vals, x_obj, ei_cons_to_vals, ei_vals_to_cons,
           ei_vals_to_obj, ei_obj_to_vals, ei_cons_to_obj, ei_obj_to_cons):
    rels = [
        ("cv", 0, ei_cons_to_vals),
        ("ov", 2, ei_obj_to_vals),
        ("vc", 1, ei_vals_to_cons),
        ("oc", 2, ei_obj_to_cons),
        ("vo", 1, ei_vals_to_obj),
        ("co", 0, ei_cons_to_obj),
    ]
    idx_arrays = []
    rows_list = []
    for _, _, ei in rels:
        *arrs, rows = _prep_edges(ei)
        idx_arrays += arrs
        rows_list.append(rows)
    table_ids = [t for _, t, _ in rels]
    agg = _make_agg(tuple(rows_list), tuple(table_ids))

    x = {
        "cons": _encode(_pad_rows(x_cons, NT), params["enc_cons"]),
        "vals": _encode(_pad_rows(x_vals, NT), params["enc_vals"]),
        "obj": _encode(_pad_rows(x_obj, NT), params["enc_obj"]),
    }

    def layer(carry, _):
        xs = carry
        tabs = [_prep_tables(*xs[t]).reshape(2 * NT, HID)
                for t in ("cons", "vals", "obj")]
        accs = agg(*tabs, *idx_arrays)
        accs = [a.reshape(2, NT, HID) for a in accs]
        h2v, xv, gv = _combine(accs[0], accs[1], xs["vals"][0],
                               params["conv_cv"], params["conv_ov"], "batch")
        h2c, xc, gc = _combine(accs[2], accs[3], xs["cons"][0],
                               params["conv_vc"], params["conv_oc"], "batch")
        h2o, xo, go = _combine(accs[4], accs[5], xs["obj"][0],
                               params["conv_vo"], params["conv_co"], "layer")
        new = {"cons": (xc, gc), "vals": (xv, gv), "obj": (xo, go)}
        return new, (h2c, h2v)

    _, (cons, vals) = lax.scan(layer, x, None, length=NL)
    vals = _pred_head(vals, params["pred_vals_1"], params["pred_vals_2"], 2)
    cons = _pred_head(cons, params["pred_cons_1"], params["pred_cons_2"], 1)
    vals_out = jnp.transpose(vals[:, :N, :], (1, 0, 2))
    cons_out = jnp.squeeze(cons[:, :N, :], axis=-1).T
    return vals_out, cons_out
